# Initial kernel scaffold; baseline (speedup 1.0000x reference)
import jax
import jax.numpy as jnp
from jax.experimental import pallas as pl


def kernel(X, Sg, enc_h0, enc_h1, dec_h0, dec_h1):
    raise NotImplementedError("write your pallas kernel here")


# speedup vs baseline: 49.5228x; 49.5228x over previous
import jax
import jax.numpy as jnp
from jax.experimental import pallas as pl
from jax.experimental.pallas import tpu as pltpu

N = 256
T = 16
K = 3
R = 2


def _roll_time_down(a, k):
    if k == 0:
        return a
    t = a.shape[0]
    return jnp.concatenate([a[t - k:, :], a[:t - k, :]], axis=0)


def _sel_even_mat(t):
    r = jax.lax.broadcasted_iota(jnp.int32, (t // 2, t), 0)
    c = jax.lax.broadcasted_iota(jnp.int32, (t // 2, t), 1)
    return (c == 2 * r).astype(jnp.float32)


def _upsample_mat(tgt_t, cur_t):
    r = jax.lax.broadcasted_iota(jnp.int32, (tgt_t, cur_t), 0)
    c = jax.lax.broadcasted_iota(jnp.int32, (tgt_t, cur_t), 1)
    return (r == 2 * c).astype(jnp.float32)


def _apply_sg(z, sg):
    return jax.lax.dot_general(
        z, sg, (((1,), (1,)), ((), ())), preferred_element_type=jnp.float32)


def _gtconv(chans, sg, h_ref, f_out):
    f_in = len(chans)
    powers = [chans]
    for _ in range(1, K):
        powers.append([_apply_sg(z, sg) for z in powers[-1]])
    rolled = [[_roll_time_down(powers[k][i], k) for i in range(f_in)]
              for k in range(K)]
    out = []
    for o in range(f_out):
        acc = None
        for i in range(f_in):
            for k in range(K):
                term = rolled[k][i] * h_ref[o, i, k]
                acc = term if acc is None else acc + term
        out.append(acc)
    return out


def _downsample_max(chans):
    t = chans[0].shape[0]
    sel = _sel_even_mat(t)
    out = []
    for z in chans:
        pair = jnp.maximum(z, jnp.concatenate([z[1:, :], z[:1, :]], axis=0))
        out.append(jnp.dot(sel, pair, preferred_element_type=jnp.float32))
    return out


def _upsample(chans, tgt_t):
    cur_t = chans[0].shape[0]
    up = _upsample_mat(tgt_t, cur_t)
    return [jnp.dot(up, z, preferred_element_type=jnp.float32) for z in chans]


def _ae_kernel(x_ref, sg_ref, e0_ref, e1_ref, d0_ref, d1_ref, out_ref):
    sg = sg_ref[...]
    chans = [x_ref[...]]

    chans = _gtconv(chans, sg, e0_ref, 2)
    chans = [jnp.maximum(z, 0.0) for z in _downsample_max(chans)]

    chans = _gtconv(chans, sg, e1_ref, 4)
    chans = [jnp.maximum(z, 0.0) for z in _downsample_max(chans)]

    chans = [jnp.maximum(z, 0.0) for z in _upsample(chans, 8)]
    chans = _gtconv(chans, sg, d0_ref, 2)

    chans = [jnp.maximum(z, 0.0) for z in _upsample(chans, 16)]
    chans = _gtconv(chans, sg, d1_ref, 1)

    out_ref[...] = chans[0]


@jax.jit
def kernel(X, Sg, enc_h0, enc_h1, dec_h0, dec_h1):
    xt = X.T
    y = pl.pallas_call(
        _ae_kernel,
        out_shape=jax.ShapeDtypeStruct((T, N), jnp.float32),
        in_specs=[
            pl.BlockSpec(memory_space=pltpu.VMEM),
            pl.BlockSpec(memory_space=pltpu.VMEM),
            pl.BlockSpec(memory_space=pltpu.SMEM),
            pl.BlockSpec(memory_space=pltpu.SMEM),
            pl.BlockSpec(memory_space=pltpu.SMEM),
            pl.BlockSpec(memory_space=pltpu.SMEM),
        ],
        out_specs=pl.BlockSpec(memory_space=pltpu.VMEM),
    )(xt, Sg, enc_h0, enc_h1, dec_h0, dec_h1)
    return y.reshape(N * T, 1)
